# baseline (speedup 1.0000x reference)
import jax
import jax.numpy as jnp
from jax.experimental import pallas as pl
from jax.experimental.pallas import tpu as pltpu

_EPS = 1e-12
_B = 8
_JP = 2


def _netvlad_kernel(x_ref, w_ref, b_ref, c_ref, out_ref, vacc_ref, sacc_ref):
    j = pl.program_id(1)
    w16 = w_ref[...].astype(jnp.bfloat16)
    b = b_ref[...]

    for s in range(_B):
        xt16 = x_ref[s, 0].astype(jnp.bfloat16)
        logits = jax.lax.dot_general(
            w16, xt16, (((1,), (1,)), ((), ())),
            preferred_element_type=jnp.float32) + b
        m = jnp.max(logits, axis=0, keepdims=True)
        e = jnp.exp(logits - m)
        a = e / jnp.sum(e, axis=0, keepdims=True)

        part = jax.lax.dot_general(
            a.astype(jnp.bfloat16), xt16, (((1,), (0,)), ((), ())),
            preferred_element_type=jnp.float32)
        psum = jnp.sum(a, axis=1, keepdims=True)

        @pl.when(j == 0)
        def _():
            vacc_ref[s] = part
            sacc_ref[s] = psum

        @pl.when(j > 0)
        def _():
            vacc_ref[s] += part
            sacc_ref[s] += psum

    @pl.when(j == _JP - 1)
    def _():
        cent = c_ref[...]
        K, C = cent.shape
        vlads = []
        for s in range(_B):
            vlad = vacc_ref[s] - sacc_ref[s] * cent
            inorm = jnp.sqrt(jnp.sum(vlad * vlad, axis=1, keepdims=True))
            vlad = vlad / jnp.maximum(inorm, _EPS)
            gnorm = jnp.sqrt(jnp.sum(vlad * vlad))
            vlads.append(vlad / jnp.maximum(gnorm, _EPS))
        out_ref[...] = jnp.stack(vlads, axis=0).reshape(_B, K * C)


def kernel(x, conv_w, conv_b, centroids):
    N, C, H, W = x.shape
    K = centroids.shape[0]
    P = H * W
    Pc = P // _JP
    xt = x.reshape(N, C, P).transpose(0, 2, 1).reshape(N, _JP, Pc, C)
    b2 = conv_b.reshape(K, 1)

    out = pl.pallas_call(
        _netvlad_kernel,
        grid=(N // _B, _JP),
        in_specs=[
            pl.BlockSpec((_B, 1, Pc, C), lambda n, j: (n, j, 0, 0)),
            pl.BlockSpec((K, C), lambda n, j: (0, 0)),
            pl.BlockSpec((K, 1), lambda n, j: (0, 0)),
            pl.BlockSpec((K, C), lambda n, j: (0, 0)),
        ],
        out_specs=pl.BlockSpec((_B, K * C), lambda n, j: (n, 0)),
        out_shape=jax.ShapeDtypeStruct((N, K * C), jnp.float32),
        scratch_shapes=[
            pltpu.VMEM((_B, K, C), jnp.float32),
            pltpu.VMEM((_B, K, 1), jnp.float32),
        ],
        compiler_params=pltpu.CompilerParams(
            dimension_semantics=("parallel", "arbitrary"),
            vmem_limit_bytes=56 * 1024 * 1024),
    )(xt, conv_w, b2, centroids)
    return out


# speedup vs baseline: 1.5803x; 1.5803x over previous
import jax
import jax.numpy as jnp
from jax.experimental import pallas as pl
from jax.experimental.pallas import tpu as pltpu

_EPS = 1e-12
_B = 8


def _netvlad_kernel(x_ref, w_ref, b_ref, c_ref, out_ref):
    w = w_ref[...]
    b = b_ref[...]
    cent = c_ref[...]

    vlads = []
    for s in range(_B):
        xt = x_ref[s]
        xt16 = xt.astype(jnp.bfloat16)
        logits = jax.lax.dot_general(
            w.astype(jnp.bfloat16), xt16, (((1,), (1,)), ((), ())),
            preferred_element_type=jnp.float32) + b
        m = jnp.max(logits, axis=0, keepdims=True)
        e = jnp.exp(logits - m)
        a = e / jnp.sum(e, axis=0, keepdims=True)

        vlad = jax.lax.dot_general(
            a.astype(jnp.bfloat16), xt16, (((1,), (0,)), ((), ())),
            preferred_element_type=jnp.float32)
        vlad = vlad - jnp.sum(a, axis=1, keepdims=True) * cent

        inorm = jnp.sqrt(jnp.sum(vlad * vlad, axis=1, keepdims=True))
        vlad = vlad / jnp.maximum(inorm, _EPS)
        gnorm = jnp.sqrt(jnp.sum(vlad * vlad))
        vlads.append(vlad / jnp.maximum(gnorm, _EPS))
    K, C = w.shape
    out_ref[...] = jnp.stack(vlads, axis=0).reshape(_B, K * C)


def kernel(x, conv_w, conv_b, centroids):
    N, C, H, W = x.shape
    K = centroids.shape[0]
    P = H * W
    xt = x.reshape(N, C, P).transpose(0, 2, 1)
    b2 = conv_b.reshape(K, 1)

    out = pl.pallas_call(
        _netvlad_kernel,
        grid=(N // _B,),
        in_specs=[
            pl.BlockSpec((_B, P, C), lambda n: (n, 0, 0)),
            pl.BlockSpec((K, C), lambda n: (0, 0)),
            pl.BlockSpec((K, 1), lambda n: (0, 0)),
            pl.BlockSpec((K, C), lambda n: (0, 0)),
        ],
        out_specs=pl.BlockSpec((_B, K * C), lambda n: (n, 0)),
        out_shape=jax.ShapeDtypeStruct((N, K * C), jnp.float32),
        compiler_params=pltpu.CompilerParams(
            dimension_semantics=("parallel",),
            vmem_limit_bytes=56 * 1024 * 1024),
    )(xt, conv_w, b2, centroids)
    return out
